# Initial kernel scaffold; baseline (speedup 1.0000x reference)
import jax
import jax.numpy as jnp
from jax.experimental import pallas as pl


def kernel(frames):
    raise NotImplementedError("write your pallas kernel here")


# speedup vs baseline: 1.3695x; 1.3695x over previous
import jax
import jax.numpy as jnp
from jax.experimental import pallas as pl
from jax.experimental.pallas import tpu as pltpu

_ALPHA = 4


def _gather_body(idx_ref, in_ref, out_ref):
    out_ref[...] = in_ref[...]


def kernel(frames):
    C, T, H, W = frames.shape
    n = T // _ALPHA
    idx = jnp.linspace(0, T - 1, n).astype(jnp.int32)
    slow = pl.pallas_call(
        _gather_body,
        grid_spec=pltpu.PrefetchScalarGridSpec(
            num_scalar_prefetch=1,
            grid=(C, n),
            in_specs=[
                pl.BlockSpec((1, 1, H, W), lambda c, t, idx_ref: (c, idx_ref[t], 0, 0))
            ],
            out_specs=pl.BlockSpec((1, 1, H, W), lambda c, t, idx_ref: (c, t, 0, 0)),
        ),
        out_shape=jax.ShapeDtypeStruct((C, n, H, W), frames.dtype),
    )(idx, frames)
    return (slow, frames)
